# Initial kernel scaffold; baseline (speedup 1.0000x reference)
import jax
import jax.numpy as jnp
from jax.experimental import pallas as pl


def kernel(x, Wr, br, Wn, bn, w1, w2, w3):
    raise NotImplementedError("write your pallas kernel here")


# speedup vs baseline: 1.5524x; 1.5524x over previous
import functools

import jax
import jax.numpy as jnp
from jax.experimental import pallas as pl
from jax.experimental.pallas import tpu as pltpu

T = 2048
D = 768
E = 8
H = 2048
NH = 512


def _router_body(x_ref, wrn_ref, brn_ref, noise_ref, gate_ref):
    x = x_ref[...]
    lg = jnp.dot(x, wrn_ref[...], preferred_element_type=jnp.float32) + brn_ref[...]
    logits = lg[:, :E]
    nlog = lg[:, E:]
    sp = jnp.maximum(nlog, 0.0) + jnp.log1p(jnp.exp(-jnp.abs(nlog)))
    noisy = logits + noise_ref[...] * sp

    lanes = jax.lax.broadcasted_iota(jnp.int32, (T, E), 1)
    m1 = jnp.max(noisy, axis=1, keepdims=True)
    i1 = jnp.min(jnp.where(noisy == m1, lanes, E), axis=1, keepdims=True)
    masked = jnp.where(lanes == i1, -jnp.inf, noisy)
    m2 = jnp.max(masked, axis=1, keepdims=True)
    i2 = jnp.min(jnp.where(masked == m2, lanes, E), axis=1, keepdims=True)
    z = jnp.exp(m2 - m1)
    g1 = 1.0 / (1.0 + z)
    g2 = 1.0 - g1
    gate_ref[...] = jnp.where(lanes == i1, g1, 0.0) + jnp.where(lanes == i2, g2, 0.0)


def _expert_body(gate_ref, x_ref, w1_ref, w3_ref, w2_ref, out_ref):
    e = pl.program_id(0)
    nh = pl.program_id(1)
    x = x_ref[...]
    h1 = jnp.dot(x, w1_ref[0], preferred_element_type=jnp.float32)
    h3 = jnp.dot(x, w3_ref[0], preferred_element_type=jnp.float32)
    h = (h1 * jax.lax.logistic(h1)) * h3
    lanes = jax.lax.broadcasted_iota(jnp.int32, (T, E), 1)
    g = jnp.sum(jnp.where(lanes == e, gate_ref[...], 0.0), axis=1, keepdims=True)
    part = jnp.dot(h * g, w2_ref[0], preferred_element_type=jnp.float32)

    @pl.when(jnp.logical_and(e == 0, nh == 0))
    def _():
        out_ref[...] = part

    @pl.when(jnp.logical_not(jnp.logical_and(e == 0, nh == 0)))
    def _():
        out_ref[...] += part


@jax.jit
def kernel(x, Wr, br, Wn, bn, w1, w2, w3):
    xf = x.reshape(T, D)
    wrn = jnp.concatenate([Wr, Wn], axis=1)
    brn = jnp.concatenate([br, bn]).reshape(1, 2 * E)
    noise = jax.random.normal(jax.random.key(42), (1, T, E), jnp.float32)[0]

    gating = pl.pallas_call(
        _router_body,
        out_shape=jax.ShapeDtypeStruct((T, E), jnp.float32),
    )(xf, wrn, brn, noise)

    out = pl.pallas_call(
        _expert_body,
        grid=(E, H // NH),
        in_specs=[
            pl.BlockSpec((T, E), lambda e, nh: (0, 0)),
            pl.BlockSpec((T, D), lambda e, nh: (0, 0)),
            pl.BlockSpec((1, D, NH), lambda e, nh: (e, 0, nh)),
            pl.BlockSpec((1, D, NH), lambda e, nh: (e, 0, nh)),
            pl.BlockSpec((1, NH, D), lambda e, nh: (e, nh, 0)),
        ],
        out_specs=pl.BlockSpec((T, D), lambda e, nh: (0, 0)),
        out_shape=jax.ShapeDtypeStruct((T, D), jnp.float32),
    )(gating, xf, w1, w3, w2)

    return out.reshape(1, T, D)
